# baseline (speedup 1.0000x reference)
import functools

import jax
import jax.numpy as jnp
from jax import lax
from jax.experimental import pallas as pl
from jax.experimental.pallas import tpu as pltpu
from jax.experimental.pallas import tpu_sc as plsc

N = 10000
E = 320000
D_IN = 128
H = 64
OUT = 64
LSTM_H = 96
G4 = 4 * LSTM_H

BLK = 1000
N_BLKS = N // BLK

NW = 32
CH = 128
EPW_CH = 79
EPW = CH * EPW_CH
EP = NW * EPW
NPAD = 10240
HP = 128
NPT = NPAD // 16
C0 = 58
C1 = 100

_MESH = plsc.VectorSubcoreMesh(core_axis_name="c", subcore_axis_name="s")
_SC_PARAMS = pltpu.CompilerParams(needs_layout_passes=False)


def _wid():
    return lax.axis_index("s") * 2 + lax.axis_index("c")


@functools.partial(
    pl.kernel,
    mesh=_MESH,
    compiler_params=_SC_PARAMS,
    out_type=jax.ShapeDtypeStruct((NW * NPAD,), jnp.float32),
    scratch_types=[
        pltpu.VMEM((CH,), jnp.int32),
        pltpu.VMEM((CH,), jnp.float32),
        pltpu.VMEM((NPAD,), jnp.float32),
    ],
)
def _deg_sc(col_hbm, ew_hbm, out_hbm, colbuf, ewbuf, degbuf):
    wid = _wid()
    zero16 = jnp.zeros((16,), jnp.float32)

    def zbody(i, _):
        degbuf[pl.ds(i * 16, 16)] = zero16
        return 0

    lax.fori_loop(0, NPAD // 16, zbody, 0)

    def chunk(k, _):
        base = wid * EPW + k * CH
        pltpu.sync_copy(col_hbm.at[pl.ds(base, CH)], colbuf)
        pltpu.sync_copy(ew_hbm.at[pl.ds(base, CH)], ewbuf)
        for j in range(CH // 16):
            sl = pl.ds(j * 16, 16)
            plsc.addupdate_scatter(degbuf, [colbuf[sl]], ewbuf[sl])
        return 0

    lax.fori_loop(0, EPW_CH, chunk, 0)
    pltpu.sync_copy(degbuf, out_hbm.at[pl.ds(wid * NPAD, NPAD)])


@functools.partial(
    pl.kernel,
    mesh=_MESH,
    compiler_params=_SC_PARAMS,
    out_type=jax.ShapeDtypeStruct((2, NPAD, HP), jnp.float32),
    scratch_types=[
        pltpu.VMEM((3, CH), jnp.int32),
        pltpu.VMEM((CH,), jnp.float32),
        pltpu.VMEM((CH, HP), jnp.float32),
        pltpu.VMEM((N,), jnp.float32),
        pltpu.VMEM((128, HP), jnp.float32),
        pltpu.VMEM_SHARED((NPAD, HP), jnp.float32),
        pltpu.SemaphoreType.DMA,
    ],
)
def _agg_sc(xw_hbm, comb_hbm, dis_hbm, out_hbm,
            idxb, normbuf, rows, disv, zbuf, acc, sem):
    cid = lax.axis_index("c")
    sid = lax.axis_index("s")
    wid = sid * 2 + cid
    zero16 = jnp.zeros((16,), jnp.float32)

    pltpu.sync_copy(dis_hbm, disv)

    def zbody(i, _):
        for f in range(HP // 16):
            zbuf[i, pl.ds(f * 16, 16)] = zero16
        return 0

    lax.fori_loop(0, 128, zbody, 0)
    for j in range(NPT // 128):
        pltpu.sync_copy(zbuf, acc.at[pl.ds(sid * NPT + j * 128, 128)])
    plsc.subcore_barrier()

    pairbase = sid * (C0 + C1)
    start = pairbase + cid * C0
    cnt = C0 + cid * (C1 - C0)

    def chunk(k, _):
        pltpu.sync_copy(comb_hbm.at[start + k], idxb)
        gather = pltpu.async_copy(xw_hbm.at[idxb.at[0]], rows, sem)
        for j in range(CH // 16):
            sl = pl.ds(j * 16, 16)
            dr = plsc.load_gather(disv, [idxb[0, sl]])
            dc = plsc.load_gather(disv, [idxb[1, sl]])
            normbuf[sl] = dr * plsc.bitcast(idxb[2, sl], jnp.float32) * dc
        gather.wait()

        def scale(j, _):
            nj = normbuf[pl.ds(j * 16, 16)]
            for u in range(16):
                nv = nj[u]
                r = j * 16 + u
                for f in range(H // 16):
                    sl2 = pl.ds(f * 16, 16)
                    rows[r, sl2] = rows[r, sl2] * nv
            return 0

        lax.fori_loop(0, CH // 16, scale, 0)
        pltpu.sync_copy(rows, acc.at[idxb.at[1]], add=True)
        return 0

    lax.fori_loop(0, cnt, chunk, 0)
    plsc.subcore_barrier()
    pltpu.sync_copy(acc.at[pl.ds(sid * NPT, NPT)],
                    out_hbm.at[cid, pl.ds(sid * NPT, NPT)])


def _dis_body(degp_ref, dis_ref):
    d = jnp.sum(degp_ref[...], axis=0, keepdims=True) + 1.0
    dis_ref[...] = lax.rsqrt(d)


def _compute_dis(deg_partials):
    return pl.pallas_call(
        _dis_body,
        out_shape=jax.ShapeDtypeStruct((1, NPAD), jnp.float32),
    )(deg_partials)


def _mm_body(x_ref, w_ref, o_ref):
    mm = jnp.dot(x_ref[...], w_ref[...], preferred_element_type=jnp.float32)
    o_ref[...] = jnp.concatenate(
        [mm, jnp.zeros((mm.shape[0], HP - H), jnp.float32)], axis=1)


def _matmul1(x, W1):
    return pl.pallas_call(
        _mm_body,
        grid=(N_BLKS,),
        in_specs=[
            pl.BlockSpec((BLK, D_IN), lambda i: (i, 0)),
            pl.BlockSpec((D_IN, H), lambda i: (0, 0)),
        ],
        out_specs=pl.BlockSpec((BLK, HP), lambda i: (i, 0)),
        out_shape=jax.ShapeDtypeStruct((N, HP), jnp.float32),
    )(x, W1)


def _combine_mm_body(parts_ref, xw_ref, dis_ref, b_ref, w_ref, h_ref, xwn_ref):
    dis = dis_ref[...]
    h = (parts_ref[0, :, :H] + parts_ref[1, :, :H]
         + dis * dis * xw_ref[:, :H] + b_ref[...])
    h_ref[...] = h
    mm = jnp.dot(h, w_ref[...], preferred_element_type=jnp.float32)
    xwn_ref[...] = jnp.concatenate(
        [mm, jnp.zeros((mm.shape[0], HP - H), jnp.float32)], axis=1)


def _combine_and_matmul(parts, xw, dis, b, W_next):
    P = parts.shape[0]
    return pl.pallas_call(
        _combine_mm_body,
        grid=(N_BLKS,),
        in_specs=[
            pl.BlockSpec((P, BLK, HP), lambda i: (0, i, 0)),
            pl.BlockSpec((BLK, HP), lambda i: (i, 0)),
            pl.BlockSpec((BLK, 1), lambda i: (i, 0)),
            pl.BlockSpec((1, H), lambda i: (0, 0)),
            pl.BlockSpec((H, H), lambda i: (0, 0)),
        ],
        out_specs=[
            pl.BlockSpec((BLK, H), lambda i: (i, 0)),
            pl.BlockSpec((BLK, HP), lambda i: (i, 0)),
        ],
        out_shape=[
            jax.ShapeDtypeStruct((N, H), jnp.float32),
            jax.ShapeDtypeStruct((N, HP), jnp.float32),
        ],
    )(parts, xw, dis, b, W_next)


def _lstm_step(x_t, h, c, wih_t, whh_t, bias):
    g = (jnp.dot(x_t, wih_t, preferred_element_type=jnp.float32)
         + jnp.dot(h, whh_t, preferred_element_type=jnp.float32) + bias)
    i = jax.nn.sigmoid(g[:, 0:LSTM_H])
    f = jax.nn.sigmoid(g[:, LSTM_H:2 * LSTM_H])
    gg = jnp.tanh(g[:, 2 * LSTM_H:3 * LSTM_H])
    o = jax.nn.sigmoid(g[:, 3 * LSTM_H:4 * LSTM_H])
    c = f * c + i * gg
    h = o * jnp.tanh(c)
    return h, c


def _final_body(h1_ref, h2_ref, parts_ref, xw3_ref, dis_ref, b3_ref,
                wihf_ref, whhf_ref, bf_ref, wihb_ref, whhb_ref, bb_ref,
                attw_ref, attb_ref, linw_ref, linb_ref, out_ref):
    dis = dis_ref[...]
    h3 = (parts_ref[0, :, :H] + parts_ref[1, :, :H]
          + dis * dis * xw3_ref[:, :H] + b3_ref[...])

    xs = (h1_ref[...], h2_ref[...], h3)

    zeros = jnp.zeros((xs[0].shape[0], LSTM_H), jnp.float32)
    bf = bf_ref[...]
    bb = bb_ref[...]
    h = c = zeros
    outs_f = []
    for t in range(3):
        h, c = _lstm_step(xs[t], h, c, wihf_ref[...], whhf_ref[...], bf)
        outs_f.append(h)
    h = c = zeros
    outs_b = [None] * 3
    for t in (2, 1, 0):
        h, c = _lstm_step(xs[t], h, c, wihb_ref[...], whhb_ref[...], bb)
        outs_b[t] = h

    attw = attw_ref[...]
    attb = attb_ref[0, 0]
    s = [jnp.sum(outs_f[t] * attw[:, :LSTM_H], axis=1)
         + jnp.sum(outs_b[t] * attw[:, LSTM_H:], axis=1) + attb
         for t in range(3)]
    m = jnp.maximum(jnp.maximum(s[0], s[1]), s[2])
    e = [jnp.exp(s[t] - m) for t in range(3)]
    denom = e[0] + e[1] + e[2]
    jk = (xs[0] * (e[0] / denom)[:, None]
          + xs[1] * (e[1] / denom)[:, None]
          + xs[2] * (e[2] / denom)[:, None])

    y = (jnp.dot(jnp.maximum(jk, 0.0), linw_ref[...],
                 preferred_element_type=jnp.float32) + linb_ref[...])
    ym = jnp.max(y, axis=1, keepdims=True)
    ye = jnp.exp(y - ym)
    out_ref[...] = ye / jnp.sum(ye, axis=1, keepdims=True)


def _final_stage(h1, h2, parts3, xw3, dis, b3,
                 W_ih_f, W_hh_f, bias_f, W_ih_b, W_hh_b, bias_b,
                 att_w, att_b, lin_w, lin_b):
    P = parts3.shape[0]
    full = lambda shp: pl.BlockSpec(shp, lambda i: tuple(0 for _ in shp))
    return pl.pallas_call(
        _final_body,
        grid=(N_BLKS,),
        in_specs=[
            pl.BlockSpec((BLK, H), lambda i: (i, 0)),
            pl.BlockSpec((BLK, H), lambda i: (i, 0)),
            pl.BlockSpec((P, BLK, HP), lambda i: (0, i, 0)),
            pl.BlockSpec((BLK, HP), lambda i: (i, 0)),
            pl.BlockSpec((BLK, 1), lambda i: (i, 0)),
            full((1, H)),
            full((H, G4)),
            full((LSTM_H, G4)),
            full((1, G4)),
            full((H, G4)),
            full((LSTM_H, G4)),
            full((1, G4)),
            full((1, 2 * LSTM_H)),
            full((1, 1)),
            full((H, OUT)),
            full((1, OUT)),
        ],
        out_specs=pl.BlockSpec((BLK, OUT), lambda i: (i, 0)),
        out_shape=jax.ShapeDtypeStruct((N, OUT), jnp.float32),
    )(h1, h2, parts3, xw3, dis, b3,
      W_ih_f, W_hh_f, bias_f, W_ih_b, W_hh_b, bias_b,
      att_w, att_b, lin_w, lin_b)


def kernel(x, edge_index, edge_attr, W1, b1, W2, b2, W3, b3,
           W_ih_f, W_hh_f, b_ih_f, b_hh_f, W_ih_b, W_hh_b, b_ih_b, b_hh_b,
           att_w, att_b, lin_w, lin_b):
    row = edge_index[0].astype(jnp.int32)
    col = edge_index[1].astype(jnp.int32)
    ew = edge_attr

    pad = EP - E
    rowp = jnp.concatenate([row, jnp.zeros((pad,), jnp.int32)])
    colp = jnp.concatenate([col, jnp.zeros((pad,), jnp.int32)])
    ewp = jnp.concatenate([ew, jnp.zeros((pad,), jnp.float32)])
    comb = jnp.stack([rowp.reshape(NW, EPW_CH, CH),
                      colp.reshape(NW, EPW_CH, CH),
                      lax.bitcast_convert_type(ewp, jnp.int32)
                         .reshape(NW, EPW_CH, CH)], axis=2)

    deg_partials = _deg_sc(colp, ewp).reshape(NW, NPAD)
    dis_row = _compute_dis(deg_partials)
    dis1d = dis_row[0, :N]
    dis2d = dis1d.reshape(N, 1)

    b1r = b1.reshape(1, H)
    b2r = b2.reshape(1, H)
    b3r = b3.reshape(1, H)

    xw1 = _matmul1(x, W1)
    combf = comb.reshape(NW * EPW_CH, 3, CH)
    parts1 = _agg_sc(xw1, combf, dis1d)
    h1, xw2 = _combine_and_matmul(parts1, xw1, dis2d, b1r, W2)
    parts2 = _agg_sc(xw2, combf, dis1d)
    h2, xw3 = _combine_and_matmul(parts2, xw2, dis2d, b2r, W3)
    parts3 = _agg_sc(xw3, combf, dis1d)

    out = _final_stage(
        h1, h2, parts3, xw3, dis2d, b3r,
        W_ih_f.T, W_hh_f.T, (b_ih_f + b_hh_f).reshape(1, G4),
        W_ih_b.T, W_hh_b.T, (b_ih_b + b_hh_b).reshape(1, G4),
        att_w.T, att_b.reshape(1, 1), lin_w, lin_b.reshape(1, OUT))
    return out


# speedup vs baseline: 1.2395x; 1.2395x over previous
import functools

import jax
import jax.numpy as jnp
from jax import lax
from jax.experimental import pallas as pl
from jax.experimental.pallas import tpu as pltpu
from jax.experimental.pallas import tpu_sc as plsc

N = 10000
E = 320000
D_IN = 128
H = 64
OUT = 64
LSTM_H = 96
G4 = 4 * LSTM_H

BLK = 1000
N_BLKS = N // BLK

NW = 32
CH = 128
EPW_CH = 79
EPW = CH * EPW_CH
EP = NW * EPW
NPAD = 10240
HP = 128
NPT = NPAD // 16
C0 = 100
C1 = 58

_MESH = plsc.VectorSubcoreMesh(core_axis_name="c", subcore_axis_name="s")
_SC_PARAMS = pltpu.CompilerParams(needs_layout_passes=False)


def _wid():
    return lax.axis_index("s") * 2 + lax.axis_index("c")


@functools.partial(
    pl.kernel,
    mesh=_MESH,
    compiler_params=_SC_PARAMS,
    out_type=jax.ShapeDtypeStruct((NW * NPAD,), jnp.float32),
    scratch_types=[
        pltpu.VMEM((CH,), jnp.int32),
        pltpu.VMEM((CH,), jnp.float32),
        pltpu.VMEM((NPAD,), jnp.float32),
    ],
)
def _deg_sc(col_hbm, ew_hbm, out_hbm, colbuf, ewbuf, degbuf):
    wid = _wid()
    zero16 = jnp.zeros((16,), jnp.float32)

    def zbody(i, _):
        degbuf[pl.ds(i * 16, 16)] = zero16
        return 0

    lax.fori_loop(0, NPAD // 16, zbody, 0)

    def chunk(k, _):
        base = wid * EPW + k * CH
        pltpu.sync_copy(col_hbm.at[pl.ds(base, CH)], colbuf)
        pltpu.sync_copy(ew_hbm.at[pl.ds(base, CH)], ewbuf)
        for j in range(CH // 16):
            sl = pl.ds(j * 16, 16)
            plsc.addupdate_scatter(degbuf, [colbuf[sl]], ewbuf[sl])
        return 0

    lax.fori_loop(0, EPW_CH, chunk, 0)
    pltpu.sync_copy(degbuf, out_hbm.at[pl.ds(wid * NPAD, NPAD)])


@functools.partial(
    pl.kernel,
    mesh=_MESH,
    compiler_params=_SC_PARAMS,
    out_type=jax.ShapeDtypeStruct((2, NPAD, HP), jnp.float32),
    scratch_types=[
        pltpu.VMEM((3, CH), jnp.int32),
        pltpu.VMEM((CH,), jnp.float32),
        pltpu.VMEM((CH, HP), jnp.float32),
        pltpu.VMEM((N,), jnp.float32),
        pltpu.VMEM((128, HP), jnp.float32),
        pltpu.VMEM_SHARED((NPAD, HP), jnp.float32),
        pltpu.SemaphoreType.DMA,
    ],
)
def _agg_sc(xw_hbm, comb_hbm, dis_hbm, out_hbm,
            idxb, normbuf, rows, disv, zbuf, acc, sem):
    cid = lax.axis_index("c")
    sid = lax.axis_index("s")
    wid = sid * 2 + cid
    zero16 = jnp.zeros((16,), jnp.float32)

    pltpu.sync_copy(dis_hbm, disv)

    def zbody(i, _):
        for f in range(HP // 16):
            zbuf[i, pl.ds(f * 16, 16)] = zero16
        return 0

    lax.fori_loop(0, 128, zbody, 0)
    for j in range(NPT // 128):
        pltpu.sync_copy(zbuf, acc.at[pl.ds(sid * NPT + j * 128, 128)])
    plsc.subcore_barrier()

    pairbase = sid * (C0 + C1)
    start = pairbase + cid * C0
    cnt = C0 + cid * (C1 - C0)

    def chunk(k, _):
        pltpu.sync_copy(comb_hbm.at[start + k], idxb)
        gather = pltpu.async_copy(xw_hbm.at[idxb.at[0]], rows, sem)
        for j in range(CH // 16):
            sl = pl.ds(j * 16, 16)
            dr = plsc.load_gather(disv, [idxb[0, sl]])
            dc = plsc.load_gather(disv, [idxb[1, sl]])
            normbuf[sl] = dr * plsc.bitcast(idxb[2, sl], jnp.float32) * dc
        gather.wait()

        def scale(j, _):
            nj = normbuf[pl.ds(j * 16, 16)]
            for u in range(16):
                nv = nj[u]
                r = j * 16 + u
                for f in range(H // 16):
                    sl2 = pl.ds(f * 16, 16)
                    rows[r, sl2] = rows[r, sl2] * nv
            return 0

        lax.fori_loop(0, CH // 16, scale, 0)
        pltpu.sync_copy(rows, acc.at[idxb.at[1]], add=True)
        return 0

    lax.fori_loop(0, cnt, chunk, 0)
    plsc.subcore_barrier()
    pltpu.sync_copy(acc.at[pl.ds(sid * NPT, NPT)],
                    out_hbm.at[cid, pl.ds(sid * NPT, NPT)])


def _dis_body(degp_ref, dis_ref):
    d = jnp.sum(degp_ref[...], axis=0, keepdims=True) + 1.0
    dis_ref[...] = lax.rsqrt(d)


def _compute_dis(deg_partials):
    return pl.pallas_call(
        _dis_body,
        out_shape=jax.ShapeDtypeStruct((1, NPAD), jnp.float32),
    )(deg_partials)


def _mm_body(x_ref, w_ref, o_ref):
    mm = jnp.dot(x_ref[...], w_ref[...], preferred_element_type=jnp.float32)
    o_ref[...] = jnp.concatenate(
        [mm, jnp.zeros((mm.shape[0], HP - H), jnp.float32)], axis=1)


def _matmul1(x, W1):
    return pl.pallas_call(
        _mm_body,
        grid=(N_BLKS,),
        in_specs=[
            pl.BlockSpec((BLK, D_IN), lambda i: (i, 0)),
            pl.BlockSpec((D_IN, H), lambda i: (0, 0)),
        ],
        out_specs=pl.BlockSpec((BLK, HP), lambda i: (i, 0)),
        out_shape=jax.ShapeDtypeStruct((N, HP), jnp.float32),
    )(x, W1)


def _combine_mm_body(parts_ref, xw_ref, dis_ref, b_ref, w_ref, h_ref, xwn_ref):
    dis = dis_ref[...]
    h = (parts_ref[0, :, :H] + parts_ref[1, :, :H]
         + dis * dis * xw_ref[:, :H] + b_ref[...])
    h_ref[...] = h
    mm = jnp.dot(h, w_ref[...], preferred_element_type=jnp.float32)
    xwn_ref[...] = jnp.concatenate(
        [mm, jnp.zeros((mm.shape[0], HP - H), jnp.float32)], axis=1)


def _combine_and_matmul(parts, xw, dis, b, W_next):
    P = parts.shape[0]
    return pl.pallas_call(
        _combine_mm_body,
        grid=(N_BLKS,),
        in_specs=[
            pl.BlockSpec((P, BLK, HP), lambda i: (0, i, 0)),
            pl.BlockSpec((BLK, HP), lambda i: (i, 0)),
            pl.BlockSpec((BLK, 1), lambda i: (i, 0)),
            pl.BlockSpec((1, H), lambda i: (0, 0)),
            pl.BlockSpec((H, H), lambda i: (0, 0)),
        ],
        out_specs=[
            pl.BlockSpec((BLK, H), lambda i: (i, 0)),
            pl.BlockSpec((BLK, HP), lambda i: (i, 0)),
        ],
        out_shape=[
            jax.ShapeDtypeStruct((N, H), jnp.float32),
            jax.ShapeDtypeStruct((N, HP), jnp.float32),
        ],
    )(parts, xw, dis, b, W_next)


def _lstm_step(x_t, h, c, wih_t, whh_t, bias):
    g = (jnp.dot(x_t, wih_t, preferred_element_type=jnp.float32)
         + jnp.dot(h, whh_t, preferred_element_type=jnp.float32) + bias)
    i = jax.nn.sigmoid(g[:, 0:LSTM_H])
    f = jax.nn.sigmoid(g[:, LSTM_H:2 * LSTM_H])
    gg = jnp.tanh(g[:, 2 * LSTM_H:3 * LSTM_H])
    o = jax.nn.sigmoid(g[:, 3 * LSTM_H:4 * LSTM_H])
    c = f * c + i * gg
    h = o * jnp.tanh(c)
    return h, c


def _final_body(h1_ref, h2_ref, parts_ref, xw3_ref, dis_ref, b3_ref,
                wihf_ref, whhf_ref, bf_ref, wihb_ref, whhb_ref, bb_ref,
                attw_ref, attb_ref, linw_ref, linb_ref, out_ref):
    dis = dis_ref[...]
    h3 = (parts_ref[0, :, :H] + parts_ref[1, :, :H]
          + dis * dis * xw3_ref[:, :H] + b3_ref[...])

    xs = (h1_ref[...], h2_ref[...], h3)

    zeros = jnp.zeros((xs[0].shape[0], LSTM_H), jnp.float32)
    bf = bf_ref[...]
    bb = bb_ref[...]
    h = c = zeros
    outs_f = []
    for t in range(3):
        h, c = _lstm_step(xs[t], h, c, wihf_ref[...], whhf_ref[...], bf)
        outs_f.append(h)
    h = c = zeros
    outs_b = [None] * 3
    for t in (2, 1, 0):
        h, c = _lstm_step(xs[t], h, c, wihb_ref[...], whhb_ref[...], bb)
        outs_b[t] = h

    attw = attw_ref[...]
    attb = attb_ref[0, 0]
    s = [jnp.sum(outs_f[t] * attw[:, :LSTM_H], axis=1)
         + jnp.sum(outs_b[t] * attw[:, LSTM_H:], axis=1) + attb
         for t in range(3)]
    m = jnp.maximum(jnp.maximum(s[0], s[1]), s[2])
    e = [jnp.exp(s[t] - m) for t in range(3)]
    denom = e[0] + e[1] + e[2]
    jk = (xs[0] * (e[0] / denom)[:, None]
          + xs[1] * (e[1] / denom)[:, None]
          + xs[2] * (e[2] / denom)[:, None])

    y = (jnp.dot(jnp.maximum(jk, 0.0), linw_ref[...],
                 preferred_element_type=jnp.float32) + linb_ref[...])
    ym = jnp.max(y, axis=1, keepdims=True)
    ye = jnp.exp(y - ym)
    out_ref[...] = ye / jnp.sum(ye, axis=1, keepdims=True)


def _final_stage(h1, h2, parts3, xw3, dis, b3,
                 W_ih_f, W_hh_f, bias_f, W_ih_b, W_hh_b, bias_b,
                 att_w, att_b, lin_w, lin_b):
    P = parts3.shape[0]
    full = lambda shp: pl.BlockSpec(shp, lambda i: tuple(0 for _ in shp))
    return pl.pallas_call(
        _final_body,
        grid=(N_BLKS,),
        in_specs=[
            pl.BlockSpec((BLK, H), lambda i: (i, 0)),
            pl.BlockSpec((BLK, H), lambda i: (i, 0)),
            pl.BlockSpec((P, BLK, HP), lambda i: (0, i, 0)),
            pl.BlockSpec((BLK, HP), lambda i: (i, 0)),
            pl.BlockSpec((BLK, 1), lambda i: (i, 0)),
            full((1, H)),
            full((H, G4)),
            full((LSTM_H, G4)),
            full((1, G4)),
            full((H, G4)),
            full((LSTM_H, G4)),
            full((1, G4)),
            full((1, 2 * LSTM_H)),
            full((1, 1)),
            full((H, OUT)),
            full((1, OUT)),
        ],
        out_specs=pl.BlockSpec((BLK, OUT), lambda i: (i, 0)),
        out_shape=jax.ShapeDtypeStruct((N, OUT), jnp.float32),
    )(h1, h2, parts3, xw3, dis, b3,
      W_ih_f, W_hh_f, bias_f, W_ih_b, W_hh_b, bias_b,
      att_w, att_b, lin_w, lin_b)


def kernel(x, edge_index, edge_attr, W1, b1, W2, b2, W3, b3,
           W_ih_f, W_hh_f, b_ih_f, b_hh_f, W_ih_b, W_hh_b, b_ih_b, b_hh_b,
           att_w, att_b, lin_w, lin_b):
    row = edge_index[0].astype(jnp.int32)
    col = edge_index[1].astype(jnp.int32)
    ew = edge_attr

    pad = EP - E
    rowp = jnp.concatenate([row, jnp.zeros((pad,), jnp.int32)])
    colp = jnp.concatenate([col, jnp.zeros((pad,), jnp.int32)])
    ewp = jnp.concatenate([ew, jnp.zeros((pad,), jnp.float32)])
    comb = jnp.stack([rowp.reshape(NW, EPW_CH, CH),
                      colp.reshape(NW, EPW_CH, CH),
                      lax.bitcast_convert_type(ewp, jnp.int32)
                         .reshape(NW, EPW_CH, CH)], axis=2)

    deg_partials = _deg_sc(colp, ewp).reshape(NW, NPAD)
    dis_row = _compute_dis(deg_partials)
    dis1d = dis_row[0, :N]
    dis2d = dis1d.reshape(N, 1)

    b1r = b1.reshape(1, H)
    b2r = b2.reshape(1, H)
    b3r = b3.reshape(1, H)

    xw1 = _matmul1(x, W1)
    combf = comb.reshape(NW * EPW_CH, 3, CH)
    parts1 = _agg_sc(xw1, combf, dis1d)
    h1, xw2 = _combine_and_matmul(parts1, xw1, dis2d, b1r, W2)
    parts2 = _agg_sc(xw2, combf, dis1d)
    h2, xw3 = _combine_and_matmul(parts2, xw2, dis2d, b2r, W3)
    parts3 = _agg_sc(xw3, combf, dis1d)

    out = _final_stage(
        h1, h2, parts3, xw3, dis2d, b3r,
        W_ih_f.T, W_hh_f.T, (b_ih_f + b_hh_f).reshape(1, G4),
        W_ih_b.T, W_hh_b.T, (b_ih_b + b_hh_b).reshape(1, G4),
        att_w.T, att_b.reshape(1, 1), lin_w, lin_b.reshape(1, OUT))
    return out
